# baseline (speedup 1.0000x reference)
import functools

import jax
import jax.numpy as jnp
from jax.experimental import pallas as pl
from jax.experimental.pallas import tpu as pltpu

H, DH, E, TOPK = 16, 64, 16, 8
EPS = 1e-6
THETA = 10000.0
NEG = -1e30


def _qkv_gate_body(x_ref, wg_ref, bg_ref, wq_ref, wk_ref, wv_ref, cosq_ref,
                   sinq_ref, cos_ref, sin_ref, gate_ref, q_ref, k_ref, v_ref):
    x = x_ref[...]
    bq = x.shape[0]
    logits = jnp.dot(x, wg_ref[...], preferred_element_type=jnp.float32)
    logits = logits + bg_ref[...]
    mx = jnp.max(logits, axis=1, keepdims=True)
    p = jnp.exp(logits - mx)
    sm = p / jnp.sum(p, axis=1, keepdims=True)
    iota = jax.lax.broadcasted_iota(jnp.int32, (bq, E), 1)
    cur = sm
    mask = jnp.zeros((bq, E), dtype=jnp.float32)
    for _ in range(TOPK):
        m = jnp.max(cur, axis=1, keepdims=True)
        cand = cur == m
        first = jnp.min(jnp.where(cand, iota, E), axis=1, keepdims=True)
        sel = iota == first
        mask = jnp.where(sel, 1.0, mask)
        cur = jnp.where(sel, -1.0, cur)
    masked = sm * mask
    gate_ref[...] = masked / (masked + EPS)
    xb = x.astype(jnp.bfloat16)
    cosq = cosq_ref[...]
    sinq = sinq_ref[...]
    cos = cos_ref[...]
    sin = sin_ref[...]
    xq = jnp.dot(xb, wq_ref[...], preferred_element_type=jnp.float32)
    xk = jnp.dot(xb, wk_ref[...], preferred_element_type=jnp.float32)
    xv = jnp.dot(xb, wv_ref[...], preferred_element_type=jnp.float32)
    half = DH // 2
    for h in range(H):
        b = h * DH
        q1 = xq[:, b:b + half]
        q2 = xq[:, b + half:b + DH]
        q_ref[h, :, :half] = (q1 * cosq - q2 * sinq).astype(jnp.bfloat16)
        q_ref[h, :, half:] = (q2 * cosq + q1 * sinq).astype(jnp.bfloat16)
        k1 = xk[:, b:b + half]
        k2 = xk[:, b + half:b + DH]
        k_ref[h, :, :half] = (k1 * cos - k2 * sin).astype(jnp.bfloat16)
        k_ref[h, :, half:] = (k2 * cos + k1 * sin).astype(jnp.bfloat16)
        v_ref[h, :, :] = xv[:, b:b + DH].astype(jnp.bfloat16)


def _attn_body(q_ref, k_ref, v_ref, g_ref, o_ref, s_ref, acc_ref, l_ref, *,
               bq, bk, nkv):
    h = pl.program_id(0)
    qi = pl.program_id(1)
    q = q_ref[0]

    def qk(j, c):
        kj = k_ref[0, pl.ds(j * bk, bk), :]
        s_ref[:, pl.ds(j * bk, bk)] = jax.lax.dot_general(
            q, kj, (((1,), (1,)), ((), ())), preferred_element_type=jnp.float32)
        return c

    jax.lax.fori_loop(0, qi, qk, 0)
    kq = k_ref[0, pl.ds(qi * bk, bk), :]
    sd = jax.lax.dot_general(q, kq, (((1,), (1,)), ((), ())),
                             preferred_element_type=jnp.float32)
    rows = jax.lax.broadcasted_iota(jnp.int32, (bq, bk), 0)
    cols = jax.lax.broadcasted_iota(jnp.int32, (bq, bk), 1)
    s_ref[:, pl.ds(qi * bk, bk)] = jnp.where(rows >= cols, sd, NEG)

    def fill(j, c):
        s_ref[:, pl.ds(j * bk, bk)] = jnp.full((bq, bk), NEG, jnp.float32)
        return c

    jax.lax.fori_loop(qi + 1, nkv, fill, 0)
    m = jnp.max(s_ref[...], axis=1, keepdims=True)
    acc_ref[...] = jnp.zeros((bq, DH), jnp.float32)
    l_ref[...] = jnp.zeros((bq, 1), jnp.float32)

    def block(j):
        sj = s_ref[:, pl.ds(j * bk, bk)]
        pj = jnp.exp(sj - m)
        vj = v_ref[0, pl.ds(j * bk, bk), :]
        return (jnp.dot(pj.astype(jnp.bfloat16), vj,
                        preferred_element_type=jnp.float32),
                jnp.sum(pj, axis=1, keepdims=True))

    def pair(p, c):
        a0, l0 = block(2 * p)
        a1, l1 = block(2 * p + 1)
        acc_ref[...] += a0 + a1
        l_ref[...] += l0 + l1
        return c

    jax.lax.fori_loop(0, (qi + 1) // 2, pair, 0)

    @pl.when((qi + 1) % 2 == 1)
    def _tail():
        a, lt = block(qi)
        acc_ref[...] += a
        l_ref[...] += lt

    ctx = acc_ref[...] / l_ref[...]
    hiota = jax.lax.broadcasted_iota(jnp.int32, (bq, E), 1)
    g = jnp.sum(jnp.where(hiota == h, g_ref[...], 0.0), axis=1, keepdims=True)
    o_ref[0] = (ctx * g).astype(jnp.bfloat16)


def _outproj_body(ctx_ref, wo_ref, o_ref):
    parts = [ctx_ref[h] for h in range(H)]
    cat = jnp.concatenate(parts, axis=1)
    o_ref[...] = jnp.dot(cat, wo_ref[...], preferred_element_type=jnp.float32)


def kernel(X, Wg, bg, Wq, Wk, Wv, Wo):
    b, s, d = X.shape
    x = X.reshape(s, d)
    bp = 512
    bq = 512
    bk = 512
    np_ = s // bp
    nq = s // bq
    half = DH // 2
    inv_freq = 1.0 / (THETA ** (jnp.arange(0, DH, 2, dtype=jnp.float32) / DH))
    t = jnp.arange(s, dtype=jnp.float32)
    freqs = jnp.outer(t, inv_freq)
    cos32 = jnp.cos(freqs)
    sin32 = jnp.sin(freqs)
    scale = 1.0 / (DH ** 0.5)
    cosq = cos32 * scale
    sinq = sin32 * scale
    bg2 = bg.reshape(1, E)
    wq_b = Wq.astype(jnp.bfloat16)
    wk_b = Wk.astype(jnp.bfloat16)
    wv_b = Wv.astype(jnp.bfloat16)
    wo_b = Wo.astype(jnp.bfloat16)

    gate, q, k, v = pl.pallas_call(
        _qkv_gate_body,
        grid=(np_,),
        in_specs=[
            pl.BlockSpec((bp, d), lambda i: (i, 0)),
            pl.BlockSpec((d, E), lambda i: (0, 0)),
            pl.BlockSpec((1, E), lambda i: (0, 0)),
            pl.BlockSpec((d, H * DH), lambda i: (0, 0)),
            pl.BlockSpec((d, H * DH), lambda i: (0, 0)),
            pl.BlockSpec((d, H * DH), lambda i: (0, 0)),
            pl.BlockSpec((bp, half), lambda i: (i, 0)),
            pl.BlockSpec((bp, half), lambda i: (i, 0)),
            pl.BlockSpec((bp, half), lambda i: (i, 0)),
            pl.BlockSpec((bp, half), lambda i: (i, 0)),
        ],
        out_specs=[
            pl.BlockSpec((bp, E), lambda i: (i, 0)),
            pl.BlockSpec((H, bp, DH), lambda i: (0, i, 0)),
            pl.BlockSpec((H, bp, DH), lambda i: (0, i, 0)),
            pl.BlockSpec((H, bp, DH), lambda i: (0, i, 0)),
        ],
        out_shape=[
            jax.ShapeDtypeStruct((s, E), jnp.float32),
            jax.ShapeDtypeStruct((H, s, DH), jnp.bfloat16),
            jax.ShapeDtypeStruct((H, s, DH), jnp.bfloat16),
            jax.ShapeDtypeStruct((H, s, DH), jnp.bfloat16),
        ],
    )(x, Wg, bg2, wq_b, wk_b, wv_b, cosq, sinq, cos32, sin32)

    ctx = pl.pallas_call(
        functools.partial(_attn_body, bq=bq, bk=bk, nkv=s // bk),
        grid=(H, nq),
        in_specs=[
            pl.BlockSpec((1, bq, DH), lambda h, i: (h, i, 0)),
            pl.BlockSpec((1, s, DH), lambda h, i: (h, 0, 0)),
            pl.BlockSpec((1, s, DH), lambda h, i: (h, 0, 0)),
            pl.BlockSpec((bq, E), lambda h, i: (i, 0)),
        ],
        out_specs=pl.BlockSpec((1, bq, DH), lambda h, i: (h, i, 0)),
        out_shape=jax.ShapeDtypeStruct((H, s, DH), jnp.bfloat16),
        scratch_shapes=[
            pltpu.VMEM((bq, s), jnp.float32),
            pltpu.VMEM((bq, DH), jnp.float32),
            pltpu.VMEM((bq, 1), jnp.float32),
        ],
    )(q, k, v, gate)

    out = pl.pallas_call(
        _outproj_body,
        grid=(np_,),
        in_specs=[
            pl.BlockSpec((H, bp, DH), lambda i: (0, i, 0)),
            pl.BlockSpec((H * DH, d), lambda i: (0, 0)),
        ],
        out_specs=pl.BlockSpec((bp, d), lambda i: (i, 0)),
        out_shape=jax.ShapeDtypeStruct((s, d), jnp.float32),
    )(ctx, wo_b)

    return out.reshape(b, s, d)


# speedup vs baseline: 1.1398x; 1.1398x over previous
import functools

import jax
import jax.numpy as jnp
from jax.experimental import pallas as pl
from jax.experimental.pallas import tpu as pltpu

H, DH, E, TOPK = 16, 64, 16, 8
EPS = 1e-6
THETA = 10000.0
NEG = -1e30


def _qkv_gate_body(x_ref, wg_ref, bg_ref, wq_ref, wk_ref, wv_ref, cosq_ref,
                   sinq_ref, cos_ref, sin_ref, gate_ref, q_ref, k_ref, v_ref,
                   km_ref):
    x = x_ref[...]
    bq = x.shape[0]
    logits = jnp.dot(x, wg_ref[...], preferred_element_type=jnp.float32)
    logits = logits + bg_ref[...]
    mx = jnp.max(logits, axis=1, keepdims=True)
    p = jnp.exp(logits - mx)
    sm = p / jnp.sum(p, axis=1, keepdims=True)
    iota = jax.lax.broadcasted_iota(jnp.int32, (bq, E), 1)
    cur = sm
    mask = jnp.zeros((bq, E), dtype=jnp.float32)
    for _ in range(TOPK):
        m = jnp.max(cur, axis=1, keepdims=True)
        cand = cur == m
        first = jnp.min(jnp.where(cand, iota, E), axis=1, keepdims=True)
        sel = iota == first
        mask = jnp.where(sel, 1.0, mask)
        cur = jnp.where(sel, -1.0, cur)
    masked = sm * mask
    gate_ref[...] = masked / (masked + EPS)
    xb = x.astype(jnp.bfloat16)
    cosq = cosq_ref[...]
    sinq = sinq_ref[...]
    cos = cos_ref[...]
    sin = sin_ref[...]
    xq = jnp.dot(xb, wq_ref[...], preferred_element_type=jnp.float32)
    xk = jnp.dot(xb, wk_ref[...], preferred_element_type=jnp.float32)
    xv = jnp.dot(xb, wv_ref[...], preferred_element_type=jnp.float32)
    xk2 = (xk * xk).astype(jnp.bfloat16)
    dio = jax.lax.broadcasted_iota(jnp.int32, (xk.shape[1], E), 0) // DH
    hio = jax.lax.broadcasted_iota(jnp.int32, (xk.shape[1], E), 1)
    seg = (dio == hio).astype(jnp.bfloat16)
    rn = jnp.dot(xk2, seg, preferred_element_type=jnp.float32)
    km_ref[...] = jnp.broadcast_to(jnp.max(rn, axis=0, keepdims=True), (8, E))
    half = DH // 2
    for h in range(H):
        b = h * DH
        q1 = xq[:, b:b + half]
        q2 = xq[:, b + half:b + DH]
        q_ref[h, :, :half] = (q1 * cosq - q2 * sinq).astype(jnp.bfloat16)
        q_ref[h, :, half:] = (q2 * cosq + q1 * sinq).astype(jnp.bfloat16)
        k1 = xk[:, b:b + half]
        k2 = xk[:, b + half:b + DH]
        k_ref[h, :, :half] = (k1 * cos - k2 * sin).astype(jnp.bfloat16)
        k_ref[h, :, half:] = (k2 * cos + k1 * sin).astype(jnp.bfloat16)
        v_ref[h, :, :] = xv[:, b:b + DH].astype(jnp.bfloat16)


def _attn_body(q_ref, k_ref, v_ref, g_ref, km_ref, o_ref, acc_ref, l_ref, *,
               bq, bk):
    h = pl.program_id(0)
    qi = pl.program_id(1)
    q = q_ref[0]
    qf = q.astype(jnp.float32)
    qn = jnp.sqrt(jnp.sum(qf * qf, axis=1, keepdims=True))
    kcol = jnp.max(km_ref[...], axis=0, keepdims=True)
    hio1 = jax.lax.broadcasted_iota(jnp.int32, (1, E), 1)
    kn2 = jnp.sum(jnp.where(hio1 == h, kcol, 0.0))
    m = qn * (jnp.sqrt(kn2) * 1.05) + 0.5
    acc_ref[...] = jnp.zeros((bq, DH), jnp.float32)
    l_ref[...] = jnp.zeros((bq, 1), jnp.float32)

    def block(j):
        kj = k_ref[0, pl.ds(j * bk, bk), :]
        s = jax.lax.dot_general(q, kj, (((1,), (1,)), ((), ())),
                                preferred_element_type=jnp.float32)
        p = jnp.exp(s - m)
        vj = v_ref[0, pl.ds(j * bk, bk), :]
        return (jnp.dot(p.astype(jnp.bfloat16), vj,
                        preferred_element_type=jnp.float32),
                jnp.sum(p, axis=1, keepdims=True))

    def pair(pi, c):
        a0, l0 = block(2 * pi)
        a1, l1 = block(2 * pi + 1)
        acc_ref[...] += a0 + a1
        l_ref[...] += l0 + l1
        return c

    jax.lax.fori_loop(0, qi // 2, pair, 0)

    @pl.when(qi % 2 == 1)
    def _tail():
        a, lt = block(qi - 1)
        acc_ref[...] += a
        l_ref[...] += lt

    kq = k_ref[0, pl.ds(qi * bk, bk), :]
    sd = jax.lax.dot_general(q, kq, (((1,), (1,)), ((), ())),
                             preferred_element_type=jnp.float32)
    rows = jax.lax.broadcasted_iota(jnp.int32, (bq, bk), 0)
    cols = jax.lax.broadcasted_iota(jnp.int32, (bq, bk), 1)
    sd = jnp.where(rows >= cols, sd, NEG)
    pd = jnp.exp(sd - m)
    vq = v_ref[0, pl.ds(qi * bk, bk), :]
    acc = acc_ref[...] + jnp.dot(pd.astype(jnp.bfloat16), vq,
                                 preferred_element_type=jnp.float32)
    l = l_ref[...] + jnp.sum(pd, axis=1, keepdims=True)

    ctx = acc / l
    hiota = jax.lax.broadcasted_iota(jnp.int32, (bq, E), 1)
    g = jnp.sum(jnp.where(hiota == h, g_ref[...], 0.0), axis=1, keepdims=True)
    o_ref[0] = (ctx * g).astype(jnp.bfloat16)


def _outproj_body(ctx_ref, wo_ref, o_ref):
    parts = [ctx_ref[h] for h in range(H)]
    cat = jnp.concatenate(parts, axis=1)
    o_ref[...] = jnp.dot(cat, wo_ref[...], preferred_element_type=jnp.float32)


def kernel(X, Wg, bg, Wq, Wk, Wv, Wo):
    b, s, d = X.shape
    x = X.reshape(s, d)
    bp = 512
    bq = 512
    bk = 512
    np_ = s // bp
    nq = s // bq
    half = DH // 2
    inv_freq = 1.0 / (THETA ** (jnp.arange(0, DH, 2, dtype=jnp.float32) / DH))
    t = jnp.arange(s, dtype=jnp.float32)
    freqs = jnp.outer(t, inv_freq)
    cos32 = jnp.cos(freqs)
    sin32 = jnp.sin(freqs)
    scale = 1.0 / (DH ** 0.5)
    cosq = cos32 * scale
    sinq = sin32 * scale
    bg2 = bg.reshape(1, E)
    wq_b = Wq.astype(jnp.bfloat16)
    wk_b = Wk.astype(jnp.bfloat16)
    wv_b = Wv.astype(jnp.bfloat16)
    wo_b = Wo.astype(jnp.bfloat16)

    gate, q, k, v, km = pl.pallas_call(
        _qkv_gate_body,
        grid=(np_,),
        in_specs=[
            pl.BlockSpec((bp, d), lambda i: (i, 0)),
            pl.BlockSpec((d, E), lambda i: (0, 0)),
            pl.BlockSpec((1, E), lambda i: (0, 0)),
            pl.BlockSpec((d, H * DH), lambda i: (0, 0)),
            pl.BlockSpec((d, H * DH), lambda i: (0, 0)),
            pl.BlockSpec((d, H * DH), lambda i: (0, 0)),
            pl.BlockSpec((bp, half), lambda i: (i, 0)),
            pl.BlockSpec((bp, half), lambda i: (i, 0)),
            pl.BlockSpec((bp, half), lambda i: (i, 0)),
            pl.BlockSpec((bp, half), lambda i: (i, 0)),
        ],
        out_specs=[
            pl.BlockSpec((bp, E), lambda i: (i, 0)),
            pl.BlockSpec((H, bp, DH), lambda i: (0, i, 0)),
            pl.BlockSpec((H, bp, DH), lambda i: (0, i, 0)),
            pl.BlockSpec((H, bp, DH), lambda i: (0, i, 0)),
            pl.BlockSpec((8, E), lambda i: (i, 0)),
        ],
        out_shape=[
            jax.ShapeDtypeStruct((s, E), jnp.float32),
            jax.ShapeDtypeStruct((H, s, DH), jnp.bfloat16),
            jax.ShapeDtypeStruct((H, s, DH), jnp.bfloat16),
            jax.ShapeDtypeStruct((H, s, DH), jnp.bfloat16),
            jax.ShapeDtypeStruct((np_ * 8, E), jnp.float32),
        ],
    )(x, Wg, bg2, wq_b, wk_b, wv_b, cosq, sinq, cos32, sin32)

    ctx = pl.pallas_call(
        functools.partial(_attn_body, bq=bq, bk=bk),
        grid=(H, nq),
        in_specs=[
            pl.BlockSpec((1, bq, DH), lambda h, i: (h, i, 0)),
            pl.BlockSpec((1, s, DH), lambda h, i: (h, 0, 0)),
            pl.BlockSpec((1, s, DH), lambda h, i: (h, 0, 0)),
            pl.BlockSpec((bq, E), lambda h, i: (i, 0)),
            pl.BlockSpec((np_ * 8, E), lambda h, i: (0, 0)),
        ],
        out_specs=pl.BlockSpec((1, bq, DH), lambda h, i: (h, i, 0)),
        out_shape=jax.ShapeDtypeStruct((H, s, DH), jnp.bfloat16),
        scratch_shapes=[
            pltpu.VMEM((bq, DH), jnp.float32),
            pltpu.VMEM((bq, 1), jnp.float32),
        ],
    )(q, k, v, gate, km)

    out = pl.pallas_call(
        _outproj_body,
        grid=(np_,),
        in_specs=[
            pl.BlockSpec((H, bp, DH), lambda i: (0, i, 0)),
            pl.BlockSpec((H * DH, d), lambda i: (0, 0)),
        ],
        out_specs=pl.BlockSpec((bp, d), lambda i: (i, 0)),
        out_shape=jax.ShapeDtypeStruct((s, d), jnp.float32),
    )(ctx, wo_b)

    return out.reshape(b, s, d)


# speedup vs baseline: 1.3459x; 1.1808x over previous
import functools

import jax
import jax.numpy as jnp
import numpy as np
from jax.experimental import pallas as pl

H, DH, E, TOPK = 16, 64, 16, 8
EPS = 1e-6
THETA = 10000.0
NEG = -1e30
HALF = DH // 2


def _qkv_gate_body(x_ref, wg_ref, bg_ref, wq_ref, wk_ref, wv_ref, cosq_ref,
                   sinq_ref, cos_ref, sin_ref, gate_ref, q1_ref, q2_ref,
                   k1_ref, k2_ref, v_ref, km_ref):
    x = x_ref[...]
    bq = x.shape[0]
    d2 = q1_ref.shape[1]
    logits = jnp.dot(x, wg_ref[...], preferred_element_type=jnp.float32)
    logits = logits + bg_ref[...]
    mx = jnp.max(logits, axis=1, keepdims=True)
    p = jnp.exp(logits - mx)
    sm = p / jnp.sum(p, axis=1, keepdims=True)
    iota = jax.lax.broadcasted_iota(jnp.int32, (bq, E), 1)
    cur = sm
    mask = jnp.zeros((bq, E), dtype=jnp.float32)
    for _ in range(TOPK):
        m = jnp.max(cur, axis=1, keepdims=True)
        cand = cur == m
        first = jnp.min(jnp.where(cand, iota, E), axis=1, keepdims=True)
        sel = iota == first
        mask = jnp.where(sel, 1.0, mask)
        cur = jnp.where(sel, -1.0, cur)
    masked = sm * mask
    gate_ref[...] = masked / (masked + EPS)
    xb = x.astype(jnp.bfloat16)
    xq = jnp.dot(xb, wq_ref[...], preferred_element_type=jnp.float32)
    xk = jnp.dot(xb, wk_ref[...], preferred_element_type=jnp.float32)
    xv = jnp.dot(xb, wv_ref[...], preferred_element_type=jnp.float32)
    cq = jnp.concatenate([cosq_ref[...]] * H, axis=1)
    sq = jnp.concatenate([sinq_ref[...]] * H, axis=1)
    ct = jnp.concatenate([cos_ref[...]] * H, axis=1)
    st = jnp.concatenate([sin_ref[...]] * H, axis=1)
    q1 = xq[:, :d2]
    q2 = xq[:, d2:]
    q1_ref[...] = (q1 * cq - q2 * sq).astype(jnp.bfloat16)
    q2_ref[...] = (q2 * cq + q1 * sq).astype(jnp.bfloat16)
    k1 = xk[:, :d2]
    k2 = xk[:, d2:]
    k1_ref[...] = (k1 * ct - k2 * st).astype(jnp.bfloat16)
    k2_ref[...] = (k2 * ct + k1 * st).astype(jnp.bfloat16)
    xk2 = (xk * xk).astype(jnp.bfloat16)
    dio = jax.lax.broadcasted_iota(jnp.int32, (2 * d2, E), 0)
    hio = jax.lax.broadcasted_iota(jnp.int32, (2 * d2, E), 1)
    seg = ((dio % d2) // HALF == hio).astype(jnp.bfloat16)
    rn = jnp.dot(xk2, seg, preferred_element_type=jnp.float32)
    km_ref[...] = jnp.broadcast_to(jnp.max(rn, axis=0, keepdims=True), (8, E))
    xvb = xv.astype(jnp.bfloat16)
    ones1 = jnp.ones((bq, 1), dtype=jnp.bfloat16)
    for h in range(H):
        v_ref[h, :, :DH] = xvb[:, h * DH:(h + 1) * DH]
        v_ref[h, :, DH:] = ones1


def _attn_body(q1_ref, q2_ref, k1_ref, k2_ref, v_ref, g_ref, km_ref, o_ref,
               *, bk, nb):
    h = pl.program_id(0)
    q = jnp.concatenate([q1_ref[0], q2_ref[0]], axis=1)
    k = jnp.concatenate([k1_ref[0], k2_ref[0]], axis=1)
    qf = q.astype(jnp.float32)
    qn = jnp.sqrt(jnp.sum(qf * qf, axis=1, keepdims=True))
    kcol = jnp.max(km_ref[...], axis=0, keepdims=True)
    hio1 = jax.lax.broadcasted_iota(jnp.int32, (1, E), 1)
    kn2 = jnp.sum(jnp.where(hio1 == h, kcol, 0.0))
    m = qn * (jnp.sqrt(kn2) * 1.05) + 1.0
    rows = jax.lax.broadcasted_iota(jnp.int32, (bk, bk), 0)
    cols = jax.lax.broadcasted_iota(jnp.int32, (bk, bk), 1)
    relmask = rows >= cols
    hiota = jax.lax.broadcasted_iota(jnp.int32, (bk, E), 1)
    for i in range(nb):
        r0 = i * bk
        qi_s = q[r0:r0 + bk, :]
        mi = m[r0:r0 + bk, :]
        acc = None
        for j in range(i + 1):
            kj = k[j * bk:(j + 1) * bk, :]
            sblk = jax.lax.dot_general(qi_s, kj, (((1,), (1,)), ((), ())),
                                       preferred_element_type=jnp.float32)
            if i == j:
                sblk = jnp.where(relmask, sblk, NEG)
            p = jnp.exp2(sblk - mi)
            vj = v_ref[0, j * bk:(j + 1) * bk, :]
            d = jnp.dot(p.astype(jnp.bfloat16), vj,
                        preferred_element_type=jnp.float32)
            acc = d if acc is None else acc + d
        ctx = acc[:, :DH] / acc[:, DH:DH + 1]
        g = jnp.sum(jnp.where(hiota == h, g_ref[r0:r0 + bk, :], 0.0),
                    axis=1, keepdims=True)
        o_ref[0, r0:r0 + bk, :] = (ctx * g).astype(jnp.bfloat16)


def _outproj_body(ctx_ref, wo_ref, o_ref):
    parts = [ctx_ref[h] for h in range(H)]
    cat = jnp.concatenate(parts, axis=1)
    o_ref[...] = jnp.dot(cat, wo_ref[...], preferred_element_type=jnp.float32)


def kernel(X, Wg, bg, Wq, Wk, Wv, Wo):
    b, s, d = X.shape
    x = X.reshape(s, d)
    bp = 512
    bk = 512
    np_ = s // bp
    d2 = H * HALF
    inv_freq = 1.0 / (THETA ** (jnp.arange(0, DH, 2, dtype=jnp.float32) / DH))
    t = jnp.arange(s, dtype=jnp.float32)
    freqs = jnp.outer(t, inv_freq)
    cos32 = jnp.cos(freqs)
    sin32 = jnp.sin(freqs)
    scale = 1.4426950408889634 / (DH ** 0.5)
    cosq = cos32 * scale
    sinq = sin32 * scale
    bg2 = bg.reshape(1, E)
    perm = np.concatenate([
        (np.arange(H)[:, None] * DH + np.arange(HALF)[None, :]).reshape(-1),
        (np.arange(H)[:, None] * DH + HALF + np.arange(HALF)[None, :]).reshape(-1),
    ])
    wq_b = Wq[:, perm].astype(jnp.bfloat16)
    wk_b = Wk[:, perm].astype(jnp.bfloat16)
    wv_b = Wv.astype(jnp.bfloat16)
    wo_b = Wo.astype(jnp.bfloat16)

    gate, q1, q2, k1, k2, v, km = pl.pallas_call(
        _qkv_gate_body,
        grid=(np_,),
        in_specs=[
            pl.BlockSpec((bp, d), lambda i: (i, 0)),
            pl.BlockSpec((d, E), lambda i: (0, 0)),
            pl.BlockSpec((1, E), lambda i: (0, 0)),
            pl.BlockSpec((d, H * DH), lambda i: (0, 0)),
            pl.BlockSpec((d, H * DH), lambda i: (0, 0)),
            pl.BlockSpec((d, H * DH), lambda i: (0, 0)),
            pl.BlockSpec((bp, HALF), lambda i: (i, 0)),
            pl.BlockSpec((bp, HALF), lambda i: (i, 0)),
            pl.BlockSpec((bp, HALF), lambda i: (i, 0)),
            pl.BlockSpec((bp, HALF), lambda i: (i, 0)),
        ],
        out_specs=[
            pl.BlockSpec((bp, E), lambda i: (i, 0)),
            pl.BlockSpec((bp, d2), lambda i: (i, 0)),
            pl.BlockSpec((bp, d2), lambda i: (i, 0)),
            pl.BlockSpec((bp, d2), lambda i: (i, 0)),
            pl.BlockSpec((bp, d2), lambda i: (i, 0)),
            pl.BlockSpec((H, bp, DH + 1), lambda i: (0, i, 0)),
            pl.BlockSpec((8, E), lambda i: (i, 0)),
        ],
        out_shape=[
            jax.ShapeDtypeStruct((s, E), jnp.float32),
            jax.ShapeDtypeStruct((s, d2), jnp.bfloat16),
            jax.ShapeDtypeStruct((s, d2), jnp.bfloat16),
            jax.ShapeDtypeStruct((s, d2), jnp.bfloat16),
            jax.ShapeDtypeStruct((s, d2), jnp.bfloat16),
            jax.ShapeDtypeStruct((H, s, DH + 1), jnp.bfloat16),
            jax.ShapeDtypeStruct((np_ * 8, E), jnp.float32),
        ],
    )(x, Wg, bg2, wq_b, wk_b, wv_b, cosq, sinq, cos32, sin32)

    q1t = q1.reshape(s, H, HALF).transpose(1, 0, 2)
    q2t = q2.reshape(s, H, HALF).transpose(1, 0, 2)
    k1t = k1.reshape(s, H, HALF).transpose(1, 0, 2)
    k2t = k2.reshape(s, H, HALF).transpose(1, 0, 2)

    ctx = pl.pallas_call(
        functools.partial(_attn_body, bk=bk, nb=s // bk),
        grid=(H,),
        in_specs=[
            pl.BlockSpec((1, s, HALF), lambda h: (h, 0, 0)),
            pl.BlockSpec((1, s, HALF), lambda h: (h, 0, 0)),
            pl.BlockSpec((1, s, HALF), lambda h: (h, 0, 0)),
            pl.BlockSpec((1, s, HALF), lambda h: (h, 0, 0)),
            pl.BlockSpec((1, s, DH + 1), lambda h: (h, 0, 0)),
            pl.BlockSpec((s, E), lambda h: (0, 0)),
            pl.BlockSpec((np_ * 8, E), lambda h: (0, 0)),
        ],
        out_specs=pl.BlockSpec((1, s, DH), lambda h: (h, 0, 0)),
        out_shape=jax.ShapeDtypeStruct((H, s, DH), jnp.bfloat16),
    )(q1t, q2t, k1t, k2t, v, gate, km)

    out = pl.pallas_call(
        _outproj_body,
        grid=(np_,),
        in_specs=[
            pl.BlockSpec((H, bp, DH), lambda i: (0, i, 0)),
            pl.BlockSpec((H * DH, d), lambda i: (0, 0)),
        ],
        out_specs=pl.BlockSpec((bp, d), lambda i: (i, 0)),
        out_shape=jax.ShapeDtypeStruct((s, d), jnp.float32),
    )(ctx, wo_b)

    return out.reshape(b, s, d)


# speedup vs baseline: 1.4817x; 1.1009x over previous
import functools

import jax
import jax.numpy as jnp
import numpy as np
from jax.experimental import pallas as pl

H, DH, E, TOPK = 16, 64, 16, 8
EPS = 1e-6
THETA = 10000.0
NEG = -1e30
HALF = DH // 2


def _qkv_gate_body(x_ref, wg_ref, bg_ref, wq_ref, wk_ref, wv_ref, cosq_ref,
                   sinq_ref, cos_ref, sin_ref, gate_ref, q_ref, k_ref, v_ref,
                   km_ref):
    x = x_ref[...]
    bq = x.shape[0]
    d2 = H * HALF
    logits = jnp.dot(x, wg_ref[...], preferred_element_type=jnp.float32)
    logits = logits + bg_ref[...]
    mx = jnp.max(logits, axis=1, keepdims=True)
    p = jnp.exp(logits - mx)
    sm = p / jnp.sum(p, axis=1, keepdims=True)
    iota = jax.lax.broadcasted_iota(jnp.int32, (bq, E), 1)
    cur = sm
    mask = jnp.zeros((bq, E), dtype=jnp.float32)
    for _ in range(TOPK):
        m = jnp.max(cur, axis=1, keepdims=True)
        cand = cur == m
        first = jnp.min(jnp.where(cand, iota, E), axis=1, keepdims=True)
        sel = iota == first
        mask = jnp.where(sel, 1.0, mask)
        cur = jnp.where(sel, -1.0, cur)
    masked = sm * mask
    gate_ref[...] = masked / (masked + EPS)
    xb = x.astype(jnp.bfloat16)
    xq = jnp.dot(xb, wq_ref[...], preferred_element_type=jnp.float32)
    xk = jnp.dot(xb, wk_ref[...], preferred_element_type=jnp.float32)
    xv = jnp.dot(xb, wv_ref[...], preferred_element_type=jnp.float32)
    cq = jnp.concatenate([cosq_ref[...]] * H, axis=1)
    sq = jnp.concatenate([sinq_ref[...]] * H, axis=1)
    ct = jnp.concatenate([cos_ref[...]] * H, axis=1)
    st = jnp.concatenate([sin_ref[...]] * H, axis=1)
    q1 = xq[:, :d2]
    q2 = xq[:, d2:]
    qr1 = (q1 * cq - q2 * sq).astype(jnp.bfloat16)
    qr2 = (q2 * cq + q1 * sq).astype(jnp.bfloat16)
    k1 = xk[:, :d2]
    k2 = xk[:, d2:]
    kr1 = (k1 * ct - k2 * st).astype(jnp.bfloat16)
    kr2 = (k2 * ct + k1 * st).astype(jnp.bfloat16)
    for h in range(H):
        hs = h * HALF
        q_ref[h, :, :HALF] = qr1[:, hs:hs + HALF]
        q_ref[h, :, HALF:] = qr2[:, hs:hs + HALF]
        k_ref[h, :, :HALF] = kr1[:, hs:hs + HALF]
        k_ref[h, :, HALF:] = kr2[:, hs:hs + HALF]
    xk2 = (xk * xk).astype(jnp.bfloat16)
    dio = jax.lax.broadcasted_iota(jnp.int32, (2 * d2, E), 0)
    hio = jax.lax.broadcasted_iota(jnp.int32, (2 * d2, E), 1)
    seg = ((dio % d2) // HALF == hio).astype(jnp.bfloat16)
    rn = jnp.dot(xk2, seg, preferred_element_type=jnp.float32)
    km_ref[...] = jnp.broadcast_to(jnp.max(rn, axis=0, keepdims=True), (8, E))
    xvb = xv.astype(jnp.bfloat16)
    ones1 = jnp.ones((bq, 1), dtype=jnp.bfloat16)
    for h in range(H):
        v_ref[h, :, :DH] = xvb[:, h * DH:(h + 1) * DH]
        v_ref[h, :, DH:] = ones1


def _attn_body(q_ref, k_ref, v_ref, g_ref, km_ref, o_ref, *, bk, nb):
    h = pl.program_id(0)
    q = q_ref[0]
    k = k_ref[0]
    qf = q.astype(jnp.float32)
    qn = jnp.sqrt(jnp.sum(qf * qf, axis=1, keepdims=True))
    kcol = jnp.max(km_ref[...], axis=0, keepdims=True)
    hio1 = jax.lax.broadcasted_iota(jnp.int32, (1, E), 1)
    kn2 = jnp.sum(jnp.where(hio1 == h, kcol, 0.0))
    m = qn * (jnp.sqrt(kn2) * 1.05) + 1.0
    rows = jax.lax.broadcasted_iota(jnp.int32, (bk, bk), 0)
    cols = jax.lax.broadcasted_iota(jnp.int32, (bk, bk), 1)
    relmask = rows >= cols
    hiota = jax.lax.broadcasted_iota(jnp.int32, (bk, E), 1)
    for i in range(nb):
        r0 = i * bk
        qi_s = q[r0:r0 + bk, :]
        mi = m[r0:r0 + bk, :]
        acc = None
        for j in range(i + 1):
            kj = k[j * bk:(j + 1) * bk, :]
            sblk = jax.lax.dot_general(qi_s, kj, (((1,), (1,)), ((), ())),
                                       preferred_element_type=jnp.float32)
            if i == j:
                sblk = jnp.where(relmask, sblk, NEG)
            p = jnp.exp2(sblk - mi)
            vj = v_ref[0, j * bk:(j + 1) * bk, :]
            d = jnp.dot(p.astype(jnp.bfloat16), vj,
                        preferred_element_type=jnp.float32)
            acc = d if acc is None else acc + d
        ctx = acc[:, :DH] / acc[:, DH:DH + 1]
        g = jnp.sum(jnp.where(hiota == h, g_ref[r0:r0 + bk, :], 0.0),
                    axis=1, keepdims=True)
        o_ref[0, r0:r0 + bk, :] = (ctx * g).astype(jnp.bfloat16)


def _outproj_body(ctx_ref, wo_ref, o_ref):
    parts = [ctx_ref[h] for h in range(H)]
    cat = jnp.concatenate(parts, axis=1)
    o_ref[...] = jnp.dot(cat, wo_ref[...], preferred_element_type=jnp.float32)


def kernel(X, Wg, bg, Wq, Wk, Wv, Wo):
    b, s, d = X.shape
    x = X.reshape(s, d)
    bp = 512
    bk = 512
    np_ = s // bp
    d2 = H * HALF
    inv_freq = 1.0 / (THETA ** (jnp.arange(0, DH, 2, dtype=jnp.float32) / DH))
    t = jnp.arange(s, dtype=jnp.float32)
    freqs = jnp.outer(t, inv_freq)
    cos32 = jnp.cos(freqs)
    sin32 = jnp.sin(freqs)
    scale = 1.4426950408889634 / (DH ** 0.5)
    cosq = cos32 * scale
    sinq = sin32 * scale
    bg2 = bg.reshape(1, E)
    perm = np.concatenate([
        (np.arange(H)[:, None] * DH + np.arange(HALF)[None, :]).reshape(-1),
        (np.arange(H)[:, None] * DH + HALF + np.arange(HALF)[None, :]).reshape(-1),
    ])
    wq_b = Wq[:, perm].astype(jnp.bfloat16)
    wk_b = Wk[:, perm].astype(jnp.bfloat16)
    wv_b = Wv.astype(jnp.bfloat16)
    wo_b = Wo.astype(jnp.bfloat16)

    gate, q, k, v, km = pl.pallas_call(
        _qkv_gate_body,
        grid=(np_,),
        in_specs=[
            pl.BlockSpec((bp, d), lambda i: (i, 0)),
            pl.BlockSpec((d, E), lambda i: (0, 0)),
            pl.BlockSpec((1, E), lambda i: (0, 0)),
            pl.BlockSpec((d, H * DH), lambda i: (0, 0)),
            pl.BlockSpec((d, H * DH), lambda i: (0, 0)),
            pl.BlockSpec((d, H * DH), lambda i: (0, 0)),
            pl.BlockSpec((bp, HALF), lambda i: (i, 0)),
            pl.BlockSpec((bp, HALF), lambda i: (i, 0)),
            pl.BlockSpec((bp, HALF), lambda i: (i, 0)),
            pl.BlockSpec((bp, HALF), lambda i: (i, 0)),
        ],
        out_specs=[
            pl.BlockSpec((bp, E), lambda i: (i, 0)),
            pl.BlockSpec((H, bp, DH), lambda i: (0, i, 0)),
            pl.BlockSpec((H, bp, DH), lambda i: (0, i, 0)),
            pl.BlockSpec((H, bp, DH + 1), lambda i: (0, i, 0)),
            pl.BlockSpec((8, E), lambda i: (i, 0)),
        ],
        out_shape=[
            jax.ShapeDtypeStruct((s, E), jnp.float32),
            jax.ShapeDtypeStruct((H, s, DH), jnp.bfloat16),
            jax.ShapeDtypeStruct((H, s, DH), jnp.bfloat16),
            jax.ShapeDtypeStruct((H, s, DH + 1), jnp.bfloat16),
            jax.ShapeDtypeStruct((np_ * 8, E), jnp.float32),
        ],
    )(x, Wg, bg2, wq_b, wk_b, wv_b, cosq, sinq, cos32, sin32)

    ctx = pl.pallas_call(
        functools.partial(_attn_body, bk=bk, nb=s // bk),
        grid=(H,),
        in_specs=[
            pl.BlockSpec((1, s, DH), lambda h: (h, 0, 0)),
            pl.BlockSpec((1, s, DH), lambda h: (h, 0, 0)),
            pl.BlockSpec((1, s, DH + 1), lambda h: (h, 0, 0)),
            pl.BlockSpec((s, E), lambda h: (0, 0)),
            pl.BlockSpec((np_ * 8, E), lambda h: (0, 0)),
        ],
        out_specs=pl.BlockSpec((1, s, DH), lambda h: (h, 0, 0)),
        out_shape=jax.ShapeDtypeStruct((H, s, DH), jnp.bfloat16),
    )(q, k, v, gate, km)

    out = pl.pallas_call(
        _outproj_body,
        grid=(np_,),
        in_specs=[
            pl.BlockSpec((H, bp, DH), lambda i: (0, i, 0)),
            pl.BlockSpec((H * DH, d), lambda i: (0, 0)),
        ],
        out_specs=pl.BlockSpec((bp, d), lambda i: (i, 0)),
        out_shape=jax.ShapeDtypeStruct((s, d), jnp.float32),
    )(ctx, wo_b)

    return out.reshape(b, s, d)


# speedup vs baseline: 1.5889x; 1.0723x over previous
import functools

import jax
import jax.numpy as jnp
import numpy as np
from jax.experimental import pallas as pl

H, DH, E, TOPK = 16, 64, 16, 8
EPS = 1e-6
THETA = 10000.0
NEG = -1e30
HALF = DH // 2


def _qkv_gate_body(x_ref, wg_ref, bg_ref, wq_ref, wk_ref, wv_ref, cosq_ref,
                   sinq_ref, cos_ref, sin_ref, gate_ref, q_ref, k_ref, v_ref,
                   km_ref):
    x = x_ref[...]
    bq = x.shape[0]
    d2 = H * HALF
    logits = jnp.dot(x, wg_ref[...], preferred_element_type=jnp.float32)
    logits = logits + bg_ref[...]
    mx = jnp.max(logits, axis=1, keepdims=True)
    p = jnp.exp(logits - mx)
    sm = p / jnp.sum(p, axis=1, keepdims=True)
    iota = jax.lax.broadcasted_iota(jnp.int32, (bq, E), 1)
    cur = sm
    mask = jnp.zeros((bq, E), dtype=jnp.float32)
    for _ in range(TOPK):
        m = jnp.max(cur, axis=1, keepdims=True)
        cand = cur == m
        first = jnp.min(jnp.where(cand, iota, E), axis=1, keepdims=True)
        sel = iota == first
        mask = jnp.where(sel, 1.0, mask)
        cur = jnp.where(sel, -1.0, cur)
    masked = sm * mask
    gate_ref[...] = masked / (masked + EPS)
    xb = x.astype(jnp.bfloat16)
    xq = jnp.dot(xb, wq_ref[...], preferred_element_type=jnp.float32)
    xk = jnp.dot(xb, wk_ref[...], preferred_element_type=jnp.float32)
    xv = jnp.dot(xb, wv_ref[...], preferred_element_type=jnp.float32)
    cq = jnp.concatenate([cosq_ref[...]] * H, axis=1)
    sq = jnp.concatenate([sinq_ref[...]] * H, axis=1)
    ct = jnp.concatenate([cos_ref[...]] * H, axis=1)
    st = jnp.concatenate([sin_ref[...]] * H, axis=1)
    q1 = xq[:, :d2]
    q2 = xq[:, d2:]
    qr1 = (q1 * cq - q2 * sq).astype(jnp.bfloat16)
    qr2 = (q2 * cq + q1 * sq).astype(jnp.bfloat16)
    k1 = xk[:, :d2]
    k2 = xk[:, d2:]
    kr1 = (k1 * ct - k2 * st).astype(jnp.bfloat16)
    kr2 = (k2 * ct + k1 * st).astype(jnp.bfloat16)
    for h in range(H):
        hs = h * HALF
        q_ref[h, :, :HALF] = qr1[:, hs:hs + HALF]
        q_ref[h, :, HALF:] = qr2[:, hs:hs + HALF]
        k_ref[h, :, :HALF] = kr1[:, hs:hs + HALF]
        k_ref[h, :, HALF:] = kr2[:, hs:hs + HALF]
    xk2 = (xk * xk).astype(jnp.bfloat16)
    dio = jax.lax.broadcasted_iota(jnp.int32, (2 * d2, E), 0)
    hio = jax.lax.broadcasted_iota(jnp.int32, (2 * d2, E), 1)
    seg = ((dio % d2) // HALF == hio).astype(jnp.bfloat16)
    rn = jnp.dot(xk2, seg, preferred_element_type=jnp.float32)
    km_ref[...] = jnp.broadcast_to(jnp.max(rn, axis=0, keepdims=True), (8, E))
    xvb = xv.astype(jnp.bfloat16)
    ones1 = jnp.ones((bq, 1), dtype=jnp.bfloat16)
    for h in range(H):
        v_ref[h, :, :DH] = xvb[:, h * DH:(h + 1) * DH]
        v_ref[h, :, DH:] = ones1


def _attn_body(q_ref, k_ref, v_ref, g_ref, km_ref, o_ref, *, bk, nb):
    h = pl.program_id(0)
    q = q_ref[0]
    k = k_ref[0]
    qf = q.astype(jnp.float32)
    qn = jnp.sqrt(jnp.sum(qf * qf, axis=1, keepdims=True))
    kcol = jnp.max(km_ref[...], axis=0, keepdims=True)
    hio1 = jax.lax.broadcasted_iota(jnp.int32, (1, E), 1)
    kn2 = jnp.sum(jnp.where(hio1 == h, kcol, 0.0))
    m = qn * (jnp.sqrt(kn2) * 1.05) + 1.0
    rows = jax.lax.broadcasted_iota(jnp.int32, (bk, bk), 0)
    cols = jax.lax.broadcasted_iota(jnp.int32, (bk, bk), 1)
    relmask = rows >= cols
    hiota = jax.lax.broadcasted_iota(jnp.int32, (bk, E), 1)
    for i in range(nb):
        r0 = i * bk
        qi_s = q[r0:r0 + bk, :]
        mi = m[r0:r0 + bk, :]
        acc = None
        for j in range(i + 1):
            kj = k[j * bk:(j + 1) * bk, :]
            sblk = jax.lax.dot_general(qi_s, kj, (((1,), (1,)), ((), ())),
                                       preferred_element_type=jnp.float32)
            if i == j:
                sblk = jnp.where(relmask, sblk, NEG)
            p = jnp.exp2(sblk - mi)
            vj = v_ref[0, j * bk:(j + 1) * bk, :]
            d = jnp.dot(p.astype(jnp.bfloat16), vj,
                        preferred_element_type=jnp.float32)
            acc = d if acc is None else acc + d
        ctx = acc[:, :DH] / acc[:, DH:DH + 1]
        g = jnp.sum(jnp.where(hiota == h, g_ref[r0:r0 + bk, :], 0.0),
                    axis=1, keepdims=True)
        o_ref[0, r0:r0 + bk, :] = (ctx * g).astype(jnp.bfloat16)


def _outproj_body(ctx_ref, wo_ref, o_ref):
    parts = [ctx_ref[h] for h in range(H)]
    cat = jnp.concatenate(parts, axis=1)
    o_ref[...] = jnp.dot(cat, wo_ref[...], preferred_element_type=jnp.float32)


def kernel(X, Wg, bg, Wq, Wk, Wv, Wo):
    b, s, d = X.shape
    x = X.reshape(s, d)
    bp = 512
    bk = 512
    np_ = s // bp
    d2 = H * HALF
    inv_freq = 1.0 / (THETA ** (jnp.arange(0, DH, 2, dtype=jnp.float32) / DH))
    t = jnp.arange(s, dtype=jnp.float32)
    freqs = jnp.outer(t, inv_freq)
    cos32 = jnp.cos(freqs)
    sin32 = jnp.sin(freqs)
    scale = 1.4426950408889634 / (DH ** 0.5)
    cosq = cos32 * scale
    sinq = sin32 * scale
    bg2 = bg.reshape(1, E)
    def _halves(w):
        return (w.reshape(d, H, 2, HALF).transpose(0, 2, 1, 3)
                .reshape(d, H * DH).astype(jnp.bfloat16))

    wq_b = _halves(Wq)
    wk_b = _halves(Wk)
    wv_b = Wv.astype(jnp.bfloat16)
    wo_b = Wo.astype(jnp.bfloat16)

    gate, q, k, v, km = pl.pallas_call(
        _qkv_gate_body,
        grid=(np_,),
        in_specs=[
            pl.BlockSpec((bp, d), lambda i: (i, 0)),
            pl.BlockSpec((d, E), lambda i: (0, 0)),
            pl.BlockSpec((1, E), lambda i: (0, 0)),
            pl.BlockSpec((d, H * DH), lambda i: (0, 0)),
            pl.BlockSpec((d, H * DH), lambda i: (0, 0)),
            pl.BlockSpec((d, H * DH), lambda i: (0, 0)),
            pl.BlockSpec((bp, HALF), lambda i: (i, 0)),
            pl.BlockSpec((bp, HALF), lambda i: (i, 0)),
            pl.BlockSpec((bp, HALF), lambda i: (i, 0)),
            pl.BlockSpec((bp, HALF), lambda i: (i, 0)),
        ],
        out_specs=[
            pl.BlockSpec((bp, E), lambda i: (i, 0)),
            pl.BlockSpec((H, bp, DH), lambda i: (0, i, 0)),
            pl.BlockSpec((H, bp, DH), lambda i: (0, i, 0)),
            pl.BlockSpec((H, bp, DH + 1), lambda i: (0, i, 0)),
            pl.BlockSpec((8, E), lambda i: (i, 0)),
        ],
        out_shape=[
            jax.ShapeDtypeStruct((s, E), jnp.float32),
            jax.ShapeDtypeStruct((H, s, DH), jnp.bfloat16),
            jax.ShapeDtypeStruct((H, s, DH), jnp.bfloat16),
            jax.ShapeDtypeStruct((H, s, DH + 1), jnp.bfloat16),
            jax.ShapeDtypeStruct((np_ * 8, E), jnp.float32),
        ],
    )(x, Wg, bg2, wq_b, wk_b, wv_b, cosq, sinq, cos32, sin32)

    ctx = pl.pallas_call(
        functools.partial(_attn_body, bk=bk, nb=s // bk),
        grid=(H,),
        in_specs=[
            pl.BlockSpec((1, s, DH), lambda h: (h, 0, 0)),
            pl.BlockSpec((1, s, DH), lambda h: (h, 0, 0)),
            pl.BlockSpec((1, s, DH + 1), lambda h: (h, 0, 0)),
            pl.BlockSpec((s, E), lambda h: (0, 0)),
            pl.BlockSpec((np_ * 8, E), lambda h: (0, 0)),
        ],
        out_specs=pl.BlockSpec((1, s, DH), lambda h: (h, 0, 0)),
        out_shape=jax.ShapeDtypeStruct((H, s, DH), jnp.bfloat16),
    )(q, k, v, gate, km)

    out = pl.pallas_call(
        _outproj_body,
        grid=(np_,),
        in_specs=[
            pl.BlockSpec((H, bp, DH), lambda i: (0, i, 0)),
            pl.BlockSpec((H * DH, d), lambda i: (0, 0)),
        ],
        out_specs=pl.BlockSpec((bp, d), lambda i: (i, 0)),
        out_shape=jax.ShapeDtypeStruct((s, d), jnp.float32),
    )(ctx, wo_b)

    return out.reshape(b, s, d)
